# baseline (speedup 1.0000x reference)
import functools

import jax
import jax.numpy as jnp
from jax import lax
from jax.experimental import pallas as pl
from jax.experimental.pallas import tpu as pltpu
from jax.experimental.pallas import tpu_sc as plsc

_K = 64
_ROWS = 128
_DEPTH = 32768
_NW = 32
_RPW = _ROWS // _NW
_NV = _DEPTH // 16
_CAND_CAP = _DEPTH + 32


def _scal(v):
    return lax.squeeze(lax.slice(v, (0,), (1,)), (0,))


def _keyify(v):
    bits = lax.bitcast_convert_type(v, jnp.int32)
    m = lax.shift_right_arithmetic(bits, 31)
    return bits ^ (m & jnp.int32(0x7FFFFFFF))


def _sc_stats(x):
    mesh = plsc.VectorSubcoreMesh(
        core_axis_name="c", subcore_axis_name="s", num_cores=2, num_subcores=16)

    @functools.partial(
        pl.kernel,
        out_type=(
            jax.ShapeDtypeStruct((_NW, 16), jnp.int32),
            jax.ShapeDtypeStruct((_NW, 16), jnp.float32),
            jax.ShapeDtypeStruct((_NW, 16), jnp.float32),
        ),
        mesh=mesh,
        compiler_params=pltpu.CompilerParams(needs_layout_passes=False),
        scratch_types=[
            pltpu.VMEM((_DEPTH,), jnp.float32),
            pltpu.VMEM((4096,), jnp.int32),
            pltpu.VMEM((_CAND_CAP,), jnp.float32),
            pltpu.VMEM((_CAND_CAP,), jnp.int32),
            pltpu.VMEM((16,), jnp.int32),
            pltpu.VMEM((16,), jnp.float32),
            pltpu.VMEM((16,), jnp.float32),
        ],
    )
    def sc_kernel(x_hbm, t_hbm, xm_hbm, dn_hbm, row_v, hist_v, cand_v,
                  ckey_v, t_stage, xm_stage, dn_stage):
        wid = lax.axis_index("s") * 2 + lax.axis_index("c")
        lanes = lax.iota(jnp.int32, 16)
        zeros16 = jnp.zeros((16,), jnp.int32)
        int_min = jnp.int32(-2147483648)

        t_acc = zeros16
        xm_acc = jnp.zeros((16,), jnp.float32)
        dn_acc = jnp.zeros((16,), jnp.float32)

        for r in range(_RPW):
            row = wid * _RPW + r
            pltpu.sync_copy(x_hbm.at[row], row_v)

            def clr(i, c):
                hist_v[pl.ds(i * 16, 16)] = zeros16
                return c
            lax.fori_loop(0, 256, clr, 0)

            ones16 = jnp.ones((16,), jnp.int32)

            def h_step(i, c):
                v = row_v[pl.ds(i * 16, 16)]
                ukey = _keyify(v) ^ int_min
                bucket = lax.shift_right_logical(ukey, 20)
                plsc.addupdate_scatter(hist_v, [bucket], ones16)
                return c
            lax.fori_loop(0, _NV, h_step, 0)

            def s_step(i, carry):
                acc, jfound, accb = carry
                j = 255 - i
                h = hist_v[pl.ds(j * 16, 16)]
                s = jnp.sum(h)
                hit = jnp.logical_and(acc < _K, acc + s >= _K)
                jfound = jnp.where(hit, j, jfound)
                accb = jnp.where(hit, acc, accb)
                return acc + s, jfound, accb
            _, jf, accb = lax.fori_loop(
                0, 256, s_step, (jnp.int32(0), jnp.int32(0), jnp.int32(0)))

            hv = hist_v[pl.ds(jf * 16, 16)]
            rev = lax.rev(hv, (0,))
            cum = plsc.cumsum(rev)
            crossed = (cum + accb) >= _K
            ffs = plsc.all_reduce_ffs(crossed)
            ffs_s = _scal(ffs)
            bucket_b = jf * 16 + (15 - ffs_s)
            c_above = accb + jnp.sum(jnp.where(lanes < ffs_s, rev, 0))
            del c_above
            t_lo = lax.shift_left(bucket_b, 20) ^ int_min

            def c_step(i, off):
                v = row_v[pl.ds(i * 16, 16)]
                k = _keyify(v)
                msk = k >= t_lo
                plsc.store_compressed(cand_v.at[pl.ds(off, 16)], v, mask=msk)
                cnt = plsc.all_reduce_population_count(msk)
                return off + _scal(cnt)
            ccnt = lax.fori_loop(0, _NV, c_step, jnp.int32(0))

            nv = (ccnt + 15) // 16
            def k_step(i, c):
                ckey_v[pl.ds(i * 16, 16)] = _keyify(cand_v[pl.ds(i * 16, 16)])
                return c
            lax.fori_loop(0, nv, k_step, 0)
            ckey_v[pl.ds(ccnt, 16)] = jnp.full((16,), int_min, jnp.int32)
            cand_v[pl.ds(ccnt, 16)] = jnp.full((16,), -3.4e38, jnp.float32)

            def b_step(i, tpat):
                bit = lax.shift_left(jnp.int32(1), jnp.int32(31) - i)
                cand_pat = tpat | bit
                cand_s = cand_pat ^ int_min

                def cnt_step(j, accv):
                    kv = ckey_v[pl.ds(j * 16, 16)]
                    return accv + plsc.all_reduce_population_count(kv >= cand_s)
                accv = lax.fori_loop(0, nv, cnt_step, zeros16)
                return jnp.where(_scal(accv) >= _K, cand_pat, tpat)
            tpat = lax.fori_loop(0, 32, b_step, jnp.int32(0))
            t_s = tpat ^ int_min

            def m_step(j, mv):
                return jnp.maximum(mv, cand_v[pl.ds(j * 16, 16)])
            mv = lax.fori_loop(0, nv, m_step, jnp.full((16,), -3.4e38, jnp.float32))
            xm = jnp.max(mv)

            def d_step(j, carry):
                ev, cv = carry
                kv = ckey_v[pl.ds(j * 16, 16)]
                vv = cand_v[pl.ds(j * 16, 16)]
                msk = kv >= t_s
                ev = ev + jnp.where(msk, jnp.exp(vv - xm), 0.0)
                cv = cv + plsc.all_reduce_population_count(msk)
                return ev, cv
            ev, cv = lax.fori_loop(
                0, nv, d_step, (jnp.zeros((16,), jnp.float32), zeros16))
            s_ge = jnp.sum(ev)
            c_ge = _scal(cv)

            bt = jnp.where(t_s < 0, t_s ^ jnp.int32(0x7FFFFFFF), t_s)
            tf = lax.bitcast_convert_type(bt, jnp.float32)
            et = _scal(jnp.exp(jnp.full((16,), tf - xm, jnp.float32)))
            denom = s_ge - (c_ge - _K).astype(jnp.float32) * et

            sel = lanes == r
            t_acc = jnp.where(sel, t_s, t_acc)
            xm_acc = jnp.where(sel, xm, xm_acc)
            dn_acc = jnp.where(sel, denom, dn_acc)

        t_stage[...] = t_acc
        xm_stage[...] = xm_acc
        dn_stage[...] = dn_acc
        pltpu.sync_copy(t_stage, t_hbm.at[wid])
        pltpu.sync_copy(xm_stage, xm_hbm.at[wid])
        pltpu.sync_copy(dn_stage, dn_hbm.at[wid])

    return sc_kernel(x)


def _tc_body(x_ref, t_ref, xm_ref, dn_ref, o_ref):
    mask = jnp.int32(0x7FFFFFFF)
    x = x_ref[...]
    b = lax.bitcast_convert_type(x, jnp.int32)
    keys = jnp.where(b < 0, b ^ mask, b)
    t = t_ref[...]
    e = jnp.exp(x - xm_ref[...])
    o_ref[...] = jnp.where(keys >= t, e / dn_ref[...], 0.0)


@functools.partial(jax.jit, static_argnums=())
def kernel(inputs):
    n_rows, depth = inputs.shape
    t_w, xm_w, dn_w = _sc_stats(inputs)
    t = t_w[:, :_RPW].reshape(n_rows, 1)
    xm = xm_w[:, :_RPW].reshape(n_rows, 1)
    dn = dn_w[:, :_RPW].reshape(n_rows, 1)
    block_rows = 32
    grid = (n_rows // block_rows,)
    small = pl.BlockSpec((block_rows, 1), lambda i: (i, 0))
    return pl.pallas_call(
        _tc_body,
        grid=grid,
        in_specs=[pl.BlockSpec((block_rows, depth), lambda i: (i, 0)),
                  small, small, small],
        out_specs=pl.BlockSpec((block_rows, depth), lambda i: (i, 0)),
        out_shape=jax.ShapeDtypeStruct((n_rows, depth), jnp.float32),
    )(inputs, t, xm, dn)


# speedup vs baseline: 1.0014x; 1.0014x over previous
import functools

import jax
import jax.numpy as jnp
from jax import lax
from jax.experimental import pallas as pl
from jax.experimental.pallas import tpu as pltpu
from jax.experimental.pallas import tpu_sc as plsc

_K = 64
_ROWS = 128
_DEPTH = 32768
_NW = 32
_RPW = _ROWS // _NW
_NV = _DEPTH // 16
_CAND_CAP = _DEPTH + 32


def _scal(v):
    return lax.squeeze(lax.slice(v, (0,), (1,)), (0,))


def _sc_stats(x):
    mesh = plsc.VectorSubcoreMesh(
        core_axis_name="c", subcore_axis_name="s", num_cores=2, num_subcores=16)

    @functools.partial(
        pl.kernel,
        out_type=(
            jax.ShapeDtypeStruct((_NW, 16), jnp.int32),
            jax.ShapeDtypeStruct((_NW, 16), jnp.float32),
            jax.ShapeDtypeStruct((_NW, 16), jnp.float32),
        ),
        mesh=mesh,
        compiler_params=pltpu.CompilerParams(needs_layout_passes=False),
        scratch_types=[
            pltpu.VMEM((_DEPTH,), jnp.float32),
            pltpu.VMEM((_DEPTH,), jnp.int32),
            pltpu.VMEM((4096,), jnp.int32),
            pltpu.VMEM((256,), jnp.int32),
            pltpu.VMEM((_CAND_CAP,), jnp.int32),
            pltpu.VMEM((16,), jnp.int32),
            pltpu.VMEM((16,), jnp.float32),
            pltpu.VMEM((16,), jnp.float32),
        ],
    )
    def sc_kernel(x_hbm, t_hbm, xm_hbm, dn_hbm, row_v, ukey_v, hist_v,
                  hist2_v, cand_v, t_stage, xm_stage, dn_stage):
        wid = lax.axis_index("s") * 2 + lax.axis_index("c")
        lanes = lax.iota(jnp.int32, 16)
        zeros16 = jnp.zeros((16,), jnp.int32)
        ones16 = jnp.ones((16,), jnp.int32)
        int_min = jnp.int32(-2147483648)
        magn = jnp.int32(0x7FFFFFFF)

        t_acc = zeros16
        xm_acc = jnp.zeros((16,), jnp.float32)
        dn_acc = jnp.zeros((16,), jnp.float32)

        for r in range(_RPW):
            row = wid * _RPW + r
            pltpu.sync_copy(x_hbm.at[row], row_v)

            def clr(i, c):
                hist_v[pl.ds(i * 16, 16)] = zeros16
                return c
            lax.fori_loop(0, 256, clr, 0, unroll=8)

            def clr2(i, c):
                hist2_v[pl.ds(i * 16, 16)] = zeros16
                return c
            lax.fori_loop(0, 16, clr2, 0, unroll=8)

            def h_step(i, c):
                v = row_v[pl.ds(i * 16, 16)]
                bits = lax.bitcast_convert_type(v, jnp.int32)
                m = lax.shift_right_arithmetic(bits, 31)
                ukey = bits ^ (m | int_min)
                ukey_v[pl.ds(i * 16, 16)] = ukey
                bucket = lax.shift_right_logical(ukey, 20)
                plsc.addupdate_scatter(hist_v, [bucket], ones16)
                plsc.addupdate_scatter(
                    hist2_v, [lax.shift_right_logical(ukey, 24)], ones16)
                return c
            lax.fori_loop(0, _NV, h_step, 0, unroll=8)

            def s_step(i, carry):
                acc, jfound, accb = carry
                j = 15 - i
                s = jnp.sum(hist2_v[pl.ds(j * 16, 16)])
                hit = jnp.logical_and(acc < _K, acc + s >= _K)
                jfound = jnp.where(hit, j, jfound)
                accb = jnp.where(hit, acc, accb)
                return acc + s, jfound, accb
            _, jc, accc = lax.fori_loop(
                0, 16, s_step, (jnp.int32(0), jnp.int32(0), jnp.int32(0)))

            def _cross(vec, above):
                rev = lax.rev(vec, (0,))
                cum = plsc.cumsum(rev)
                ffs_s = _scal(plsc.all_reduce_ffs((cum + above) >= _K))
                lane = 15 - ffs_s
                c_above = above + jnp.sum(jnp.where(lanes < ffs_s, rev, 0))
                return lane, c_above

            l2, acc2 = _cross(hist2_v[pl.ds(jc * 16, 16)], accc)
            sb = jc * 16 + l2
            l1, acc1 = _cross(hist_v[pl.ds(sb * 16, 16)], acc2)
            del acc1
            bucket_b = sb * 16 + l1
            p_lo_s = lax.shift_left(bucket_b, 20) ^ int_min

            def c_step(i, off):
                ku = ukey_v[pl.ds(i * 16, 16)]
                msk = (ku ^ int_min) >= p_lo_s
                plsc.store_compressed(cand_v.at[pl.ds(off, 16)], ku, mask=msk)
                cnt = plsc.all_reduce_population_count(msk)
                return off + _scal(cnt)
            ccnt = lax.fori_loop(0, _NV, c_step, jnp.int32(0), unroll=8)

            nv = (ccnt + 15) // 16
            cand_v[pl.ds(ccnt, 16)] = zeros16

            def b_step(i, tpat):
                bit = lax.shift_left(jnp.int32(1), jnp.int32(31) - i)
                cand_s = (tpat | bit) ^ int_min

                def cnt_step(j, accv):
                    kv = cand_v[pl.ds(j * 16, 16)] ^ int_min
                    return accv + plsc.all_reduce_population_count(kv >= cand_s)
                accv = lax.fori_loop(0, nv, cnt_step, zeros16)
                return jnp.where(_scal(accv) >= _K, tpat | bit, tpat)
            tpat = lax.fori_loop(0, 32, b_step, jnp.int32(0))
            t_s = tpat ^ int_min

            def unkey(ku):
                ks = ku ^ int_min
                m2 = lax.shift_right_arithmetic(ks, 31)
                return lax.bitcast_convert_type(ks ^ (m2 & magn), jnp.float32)

            def m_step(j, mv):
                ku = cand_v[pl.ds(j * 16, 16)]
                vv = jnp.where(ku == 0, jnp.float32(-3.4e38), unkey(ku))
                return jnp.maximum(mv, vv)
            mv = lax.fori_loop(
                0, nv, m_step, jnp.full((16,), -3.4e38, jnp.float32))
            xm = jnp.max(mv)

            def d_step(j, carry):
                ev, cv = carry
                ku = cand_v[pl.ds(j * 16, 16)]
                msk = (ku ^ int_min) >= t_s
                ev = ev + jnp.where(msk, jnp.exp(unkey(ku) - xm), 0.0)
                cv = cv + plsc.all_reduce_population_count(msk)
                return ev, cv
            ev, cv = lax.fori_loop(
                0, nv, d_step, (jnp.zeros((16,), jnp.float32), zeros16))
            s_ge = jnp.sum(ev)
            c_ge = _scal(cv)

            bt = jnp.where(t_s < 0, t_s ^ magn, t_s)
            tf = lax.bitcast_convert_type(bt, jnp.float32)
            et = _scal(jnp.exp(jnp.full((16,), tf - xm, jnp.float32)))
            denom = s_ge - (c_ge - _K).astype(jnp.float32) * et

            sel = lanes == r
            t_acc = jnp.where(sel, t_s, t_acc)
            xm_acc = jnp.where(sel, xm, xm_acc)
            dn_acc = jnp.where(sel, denom, dn_acc)

        t_stage[...] = t_acc
        xm_stage[...] = xm_acc
        dn_stage[...] = dn_acc
        pltpu.sync_copy(t_stage, t_hbm.at[wid])
        pltpu.sync_copy(xm_stage, xm_hbm.at[wid])
        pltpu.sync_copy(dn_stage, dn_hbm.at[wid])

    return sc_kernel(x)


def _tc_body(x_ref, t_ref, xm_ref, dn_ref, o_ref):
    mask = jnp.int32(0x7FFFFFFF)
    x = x_ref[...]
    b = lax.bitcast_convert_type(x, jnp.int32)
    keys = jnp.where(b < 0, b ^ mask, b)
    t = t_ref[...]
    e = jnp.exp(x - xm_ref[...])
    o_ref[...] = jnp.where(keys >= t, e / dn_ref[...], 0.0)


@functools.partial(jax.jit, static_argnums=())
def kernel(inputs):
    n_rows, depth = inputs.shape
    t_w, xm_w, dn_w = _sc_stats(inputs)
    t = t_w[:, :_RPW].reshape(n_rows, 1)
    xm = xm_w[:, :_RPW].reshape(n_rows, 1)
    dn = dn_w[:, :_RPW].reshape(n_rows, 1)
    block_rows = 32
    grid = (n_rows // block_rows,)
    small = pl.BlockSpec((block_rows, 1), lambda i: (i, 0))
    return pl.pallas_call(
        _tc_body,
        grid=grid,
        in_specs=[pl.BlockSpec((block_rows, depth), lambda i: (i, 0)),
                  small, small, small],
        out_specs=pl.BlockSpec((block_rows, depth), lambda i: (i, 0)),
        out_shape=jax.ShapeDtypeStruct((n_rows, depth), jnp.float32),
    )(inputs, t, xm, dn)
